# baseline (speedup 1.0000x reference)
import jax
import jax.numpy as jnp
from jax import lax
from jax.experimental import pallas as pl
from jax.experimental.pallas import tpu as pltpu
from jax.experimental.pallas import tpu_sc as plsc

UTT_LEN = 20
VOCAB1 = 100001
BATCH = 16384
EMB = 32
NUM_ROWS = UTT_LEN * VOCAB1

TBLK = 2048
NBLK = -(-NUM_ROWS // TBLK)
PROWS = NBLK * (TBLK // 4)
ROWS2 = PROWS * 4

NC = 2
NS = 16
L = 16
NW = NC * NS
BPW = BATCH // NW
CH = 128
NCH = BPW // CH
RU = 4


def _pack_body(x_ref, o_ref):
    x3 = x_ref[...].reshape(EMB, TBLK // 4, 4)
    for k in range(4):
        o_ref[:, k * EMB:(k + 1) * EMB] = jnp.transpose(x3[:, :, k], (1, 0))


@jax.jit
def _pack(tablet):
    return pl.pallas_call(
        _pack_body,
        grid=(NBLK,),
        in_specs=[pl.BlockSpec((EMB, TBLK), lambda i: (0, i))],
        out_specs=pl.BlockSpec((TBLK // 4, 4 * EMB), lambda i: (i, 0)),
        out_shape=jax.ShapeDtypeStruct((PROWS, 4 * EMB), jnp.float32),
    )(tablet)


def _body(utts_hbm, table_hbm, out_hbm, idx_v, rows_v, acc_v, sem0, sem1):
    cid = lax.axis_index("c")
    sid = lax.axis_index("s")
    wid = sid * NC + cid
    base = wid * BPW

    for t in range(UTT_LEN):
        pltpu.sync_copy(utts_hbm.at[t, pl.ds(base, BPW)],
                        idx_v.at[pl.ds(t * BPW, BPW)])

    for t in range(1, UTT_LEN):
        off = jnp.int32(t * VOCAB1)

        def _add_off(j, _, t=t, off=off):
            sl = pl.ds(t * BPW + j * L, L)
            idx_v[sl] = idx_v[sl] + off
            return 0

        lax.fori_loop(0, BPW // L, _add_off, 0)

    sems = (sem0, sem1)

    def fire(t, dst, sem):
        handles = []
        for c in range(NCH):
            isl = idx_v.at[pl.ds(t * BPW + c * CH, CH)]
            handles.append(
                pltpu.async_copy(table_hbm.at[isl],
                                 dst.at[pl.ds(c * CH, CH)], sem))
        return handles

    def drain(handles):
        for h in handles:
            h.wait()

    def accumulate(b):
        def _acc(i, _, b=b):
            r = i * RU
            for k in range(RU):
                for h in range(2):
                    sl = pl.ds(h * L, L)
                    plsc.addupdate(acc_v.at[r + k, sl], rows_v[b, r + k, sl])
            return 0

        lax.fori_loop(0, BPW // RU, _acc, 0)

    h_acc = fire(0, acc_v, sems[0])
    h_cur = fire(1, rows_v.at[0], sems[1])
    drain(h_acc)
    for t in range(1, UTT_LEN):
        b = (t - 1) % 2
        drain(h_cur)
        if t + 1 < UTT_LEN:
            h_next = fire(t + 1, rows_v.at[t % 2], sems[(t + 1) % 2])
        accumulate(b)
        if t + 1 < UTT_LEN:
            h_cur = h_next

    pltpu.sync_copy(acc_v, out_hbm.at[pl.ds(base, BPW)])


@jax.jit
def _emb_sum(utts32, table2d):
    fn = pl.kernel(
        _body,
        out_type=jax.ShapeDtypeStruct((BATCH, EMB), jnp.float32),
        mesh=plsc.VectorSubcoreMesh(core_axis_name="c", subcore_axis_name="s",
                                    num_cores=NC, num_subcores=NS),
        scratch_types=[
            pltpu.VMEM((UTT_LEN * BPW,), jnp.int32),
            pltpu.VMEM((2, BPW, EMB), jnp.float32),
            pltpu.VMEM((BPW, EMB), jnp.float32),
            pltpu.SemaphoreType.DMA,
            pltpu.SemaphoreType.DMA,
        ],
        compiler_params=pltpu.CompilerParams(use_tc_tiling_on_sc=False),
    )
    return fn(utts32, table2d)


def kernel(utts, table):
    tablet = jnp.swapaxes(table, 0, 1)
    table2d = _pack(tablet).reshape(ROWS2, EMB)
    utts32 = utts.astype(jnp.int32)
    out = _emb_sum(utts32, table2d)
    return out.reshape(BATCH, EMB // 8, 8)


# speedup vs baseline: 12.4118x; 12.4118x over previous
import jax
import jax.numpy as jnp
from jax import lax
from jax.experimental import pallas as pl
from jax.experimental.pallas import tpu as pltpu
from jax.experimental.pallas import tpu_sc as plsc

UTT_LEN = 20
VOCAB1 = 100001
BATCH = 16384
EMB = 32
NUM_ROWS = UTT_LEN * VOCAB1

NC = 2
NS = 16
L = 16
NW = NC * NS
BPW = BATCH // NW
CH = 128
NCH = BPW // CH
RU = 4


def _body(utts_hbm, table_hbm, out_hbm, idx_v, rows_v, acc_v, sem0, sem1):
    cid = lax.axis_index("c")
    sid = lax.axis_index("s")
    wid = sid * NC + cid
    base = wid * BPW

    for t in range(UTT_LEN):
        pltpu.sync_copy(utts_hbm.at[t, pl.ds(base, BPW)],
                        idx_v.at[pl.ds(t * BPW, BPW)])

    for t in range(1, UTT_LEN):
        off = jnp.int32(t * VOCAB1)

        def _add_off(j, _, t=t, off=off):
            sl = pl.ds(t * BPW + j * L, L)
            idx_v[sl] = idx_v[sl] + off
            return 0

        lax.fori_loop(0, BPW // L, _add_off, 0)

    sems = (sem0, sem1)

    def fire(t, dst, sem):
        handles = []
        for c in range(NCH):
            isl = idx_v.at[pl.ds(t * BPW + c * CH, CH)]
            handles.append(
                pltpu.async_copy(table_hbm.at[isl],
                                 dst.at[pl.ds(c * CH, CH)], sem))
        return handles

    def drain(handles):
        for h in handles:
            h.wait()

    def accumulate(b):
        def _acc(i, _, b=b):
            r = i * RU
            for k in range(RU):
                for h in range(2):
                    sl = pl.ds(h * L, L)
                    plsc.addupdate(acc_v.at[r + k, sl], rows_v[b, r + k, sl])
            return 0

        lax.fori_loop(0, BPW // RU, _acc, 0)

    h_acc = fire(0, acc_v, sems[0])
    h_cur = fire(1, rows_v.at[0], sems[1])
    drain(h_acc)
    for t in range(1, UTT_LEN):
        b = (t - 1) % 2
        drain(h_cur)
        if t + 1 < UTT_LEN:
            h_next = fire(t + 1, rows_v.at[t % 2], sems[(t + 1) % 2])
        accumulate(b)
        if t + 1 < UTT_LEN:
            h_cur = h_next

    pltpu.sync_copy(acc_v, out_hbm.at[pl.ds(base, BPW)])


@jax.jit
def _emb_sum(utts32, table):
    fn = pl.kernel(
        _body,
        out_type=jax.ShapeDtypeStruct((BATCH, EMB), jnp.float32),
        mesh=plsc.VectorSubcoreMesh(core_axis_name="c", subcore_axis_name="s",
                                    num_cores=NC, num_subcores=NS),
        scratch_types=[
            pltpu.VMEM((UTT_LEN * BPW,), jnp.int32),
            pltpu.VMEM((2, BPW, EMB), jnp.float32),
            pltpu.VMEM((BPW, EMB), jnp.float32),
            pltpu.SemaphoreType.DMA,
            pltpu.SemaphoreType.DMA,
        ],
        compiler_params=pltpu.CompilerParams(use_tc_tiling_on_sc=False),
    )
    return fn(utts32, table)


def kernel(utts, table):
    utts32 = utts.astype(jnp.int32)
    out = _emb_sum(utts32, table)
    return out.reshape(BATCH, 4, 8)
